# baseline (speedup 1.0000x reference)
import functools

import jax
import jax.numpy as jnp
from jax import lax
from jax.experimental import pallas as pl
from jax.experimental.pallas import tpu as pltpu
from jax.experimental.pallas import tpu_sc as plsc

N = 10000
E = 320000
R = 1000
D = 128
DEPTH = 2
F3 = D * (DEPTH + 1)

_NC = 2
_NS = 16
_NW = _NC * _NS
_CH = 40
_EPW = E // _NW
_CPW = _EPW // _CH
_D2 = 144
_ZCH = 80
_NZC = N // _ZCH
_ZITER = (_NZC + _NS - 1) // _NS


def _normalize(x, axis):
    n = jnp.sqrt(jnp.sum(x * x, axis=axis, keepdims=True))
    return x / jnp.maximum(n, 1e-12)


def _sc_ids():
    cid = lax.axis_index("c")
    sid = lax.axis_index("s")
    return cid, sid, sid * _NC + cid


def _zero_acc(zeros_hbm, acc, sid):
    def zbody(j, carry):
        ch = sid + _NS * j

        @pl.when(ch < _NZC)
        def _():
            sl = pl.ds(ch * _ZCH, _ZCH)
            pltpu.sync_copy(zeros_hbm.at[sl], acc.at[sl])

        return carry

    lax.fori_loop(0, _ZITER, zbody, 0)
    plsc.subcore_barrier()


def _copy_out(acc, out, cid, sid):
    plsc.subcore_barrier()

    def obody(j, carry):
        ch = sid + _NS * j

        @pl.when(ch < _NZC)
        def _():
            sl = pl.ds(ch * _ZCH, _ZCH)
            pltpu.sync_copy(acc.at[sl], out.at[cid, sl])

        return carry

    lax.fori_loop(0, _ZITER, obody, 0)


_NB = 3
_NG = 2
_NBODY = 6
_LOOPC = (_CPW - 2) // _NBODY
_TAIL = _CPW - _NBODY * _LOOPC


def _msg_compute(xrows, rrows, orows):
    lanes = lax.iota(jnp.int32, 16)

    def gbody(g, gcarry):
        for kk in range(8):
            e = g * 8 + kk
            xs = [xrows[e, pl.ds(16 * q, 16)] for q in range(8)]
            rs = [rrows[e, pl.ds(16 * q, 16)] for q in range(9)]
            v = xs[0] * rs[0]
            for q in range(1, 8):
                v = v + xs[q] * rs[q]
            s = jnp.sum(v)
            w = rs[8][0]
            c2 = -2.0 * w * s
            for q in range(8):
                orows[e, pl.ds(16 * q, 16)] = w * xs[q] + c2 * rs[q]
            orows[e, pl.ds(D, 16)] = jnp.where(lanes == 0, w, 0.0)
        return gcarry

    lax.fori_loop(0, _CH // 8, gbody, 0)


def _copy_compute(xrows, orows):
    lanes = lax.iota(jnp.int32, 16)

    def gbody(g, gcarry):
        for kk in range(8):
            e = g * 8 + kk
            for q in range(8):
                orows[e, pl.ds(16 * q, 16)] = xrows[e, pl.ds(16 * q, 16)]
            orows[e, pl.ds(D, 16)] = jnp.where(lanes == 0, 1.0, 0.0)
        return gcarry

    lax.fori_loop(0, _CH // 8, gbody, 0)


def _msg_body(feats, reltab, sdr, mode, zeros, out, ibs, xrs, rrs, ors,
              mbuf, sgx, sgr, sss, acc):
    cid, sid, wid = _sc_ids()
    pltpu.sync_copy(mode, mbuf)
    plain = mbuf[...][0] != 0
    _zero_acc(zeros, acc, sid)

    def start_gather(t, p2, p3):
        pltpu.sync_copy(sdr.at[wid * _CPW + t], ibs[p3])
        pltpu.async_copy(feats.at[ibs[p3].at[0]], xrs[p2], sgx[p2])

        @pl.when(jnp.logical_not(plain))
        def _():
            pltpu.async_copy(reltab.at[ibs[p3].at[2]], rrs[p2], sgr[p2])

    def process(t, p2, p3, prefetch):
        def _wait_prev():
            q = (p3 + 1) % _NB
            pltpu.make_async_copy(ors[q], acc.at[ibs[q].at[1]], sss[q]).wait()

        if isinstance(t, int):
            if t >= 2:
                _wait_prev()
        else:
            pl.when(t >= 2)(_wait_prev)
        if prefetch:
            start_gather(t + 1, (p2 + 1) % _NG, (p3 + 1) % _NB)
        pltpu.make_async_copy(feats.at[ibs[p3].at[0]], xrs[p2], sgx[p2]).wait()

        @pl.when(plain)
        def _():
            _copy_compute(xrs[p2], ors[p3])

        @pl.when(jnp.logical_not(plain))
        def _():
            pltpu.make_async_copy(reltab.at[ibs[p3].at[2]], rrs[p2],
                                  sgr[p2]).wait()
            _msg_compute(xrs[p2], rrs[p2], ors[p3])

        pltpu.async_copy(ors[p3], acc.at[ibs[p3].at[1]], sss[p3], add=True)

    start_gather(0, 0, 0)

    def cbody(t6, carry):
        for b in range(_NBODY):
            process(t6 * _NBODY + b, b % _NG, b % _NB, True)
        return carry

    lax.fori_loop(0, _LOOPC, cbody, 0)
    t0 = _LOOPC * _NBODY
    for b in range(_TAIL):
        t = t0 + b
        process(t, t % _NG, t % _NB, b < _TAIL - 1)
    for t in (_CPW - 2, _CPW - 1):
        p = t % _NB
        pltpu.make_async_copy(ors[p], acc.at[ibs[p].at[1]], sss[p]).wait()
    _copy_out(acc, out, cid, sid)


_TPAD = _NW * R


def _sc_msg(feats, reltab, src, dst, rel, plain, zeros):
    def body(feats_, reltab_, sdr, mode, zeros_, out, i0, i1, i2, x0, x1,
             r0, r1, o0, o1, o2, mb, gx0, gx1, gr0, gr1,
             s0, s1, s2, acc):
        _msg_body(feats_, reltab_, sdr, mode, zeros_, out, [i0, i1, i2],
                  [x0, x1], [r0, r1], [o0, o1, o2], mb,
                  [gx0, gx1], [gr0, gr1], [s0, s1, s2], acc)

    k = functools.partial(
        pl.kernel,
        mesh=plsc.VectorSubcoreMesh(core_axis_name="c", subcore_axis_name="s"),
        out_type=jax.ShapeDtypeStruct((_NC, N, _D2), jnp.float32),
        compiler_params=pltpu.CompilerParams(use_tc_tiling_on_sc=False, needs_layout_passes=False),
        scratch_types=(
            [pltpu.VMEM((3, _CH), jnp.int32)] * _NB
            + [pltpu.VMEM((_CH, D), jnp.float32)] * _NG
            + [pltpu.VMEM((_CH, _D2), jnp.float32)] * _NG
            + [pltpu.VMEM((_CH, _D2), jnp.float32)] * _NB
            + [pltpu.VMEM((16,), jnp.int32)]
            + [pltpu.SemaphoreType.DMA] * (2 * _NG + _NB)
            + [pltpu.VMEM_SHARED((N, _D2), jnp.float32)]
        ),
    )(body)
    sdr = jnp.stack([src, dst, rel]).reshape(3, E // _CH, _CH)
    sdr = sdr.transpose(1, 0, 2)
    fpad = jnp.zeros((_TPAD, D), jnp.float32).at[:feats.shape[0]].set(feats)
    mode = jnp.full((16,), 1 if plain else 0, jnp.int32)
    return k(fpad, reltab, sdr, mode, zeros)


_PB = 1000


def _post_body(acc_ref, o_ref):
    a = acc_ref[0] + acc_ref[1]
    den = jnp.maximum(a[:, D:D + 1], 1e-30)
    o_ref[...] = jnp.tanh(a[:, :D] / den)


def _post(acc):
    return pl.pallas_call(
        _post_body,
        grid=(N // _PB,),
        in_specs=[pl.BlockSpec((_NC, _PB, _D2), lambda i: (0, i, 0))],
        out_specs=pl.BlockSpec((_PB, D), lambda i: (i, 0)),
        out_shape=jax.ShapeDtypeStruct((N, D), jnp.float32),
    )(acc)


def _tail_body(xe_ref, xr_ref, epnt_ref, ep_ref, eg_ref, eb_ref,
               rpnt_ref, rp_ref, rg_ref, rb_ref, out_ref):
    def one(x, pnt, p, g, b):
        xn = x * lax.rsqrt(jnp.maximum(jnp.sum(x * x, axis=1, keepdims=True), 1e-24))
        logits = jnp.dot(xn, pnt, preferred_element_type=jnp.float32)
        m = jnp.max(logits, axis=1, keepdims=True)
        ex = jnp.exp(logits - m)
        att = ex / jnp.sum(ex, axis=1, keepdims=True)
        pf = x - jnp.dot(att, p, preferred_element_type=jnp.float32)
        gate = jax.nn.sigmoid(jnp.dot(pf, g, preferred_element_type=jnp.float32) + b)
        return gate * x + (1.0 - gate) * pf

    out_ref[:, :F3] = one(xe_ref[...], epnt_ref[...], ep_ref[...], eg_ref[...], eb_ref[...])
    out_ref[:, F3:] = one(xr_ref[...], rpnt_ref[...], rp_ref[...], rg_ref[...], rb_ref[...])


def _tail(out_e, out_r, e_proxy, e_gate, e_bias, r_proxy, r_gate, r_bias):
    epnt = _normalize(e_proxy, axis=-1).T
    rpnt = _normalize(r_proxy, axis=-1).T
    row_spec = pl.BlockSpec((_PB, F3), lambda i: (i, 0))
    full = lambda shape: pl.BlockSpec(shape, lambda i: (0,) * len(shape))
    return pl.pallas_call(
        _tail_body,
        grid=(N // _PB,),
        in_specs=[
            row_spec, row_spec,
            full(epnt.shape), full(e_proxy.shape), full(e_gate.shape), full(e_bias.shape),
            full(rpnt.shape), full(r_proxy.shape), full(r_gate.shape), full(r_bias.shape),
        ],
        out_specs=pl.BlockSpec((_PB, 2 * F3), lambda i: (i, 0)),
        out_shape=jax.ShapeDtypeStruct((N, 2 * F3), jnp.float32),
    )(out_e, out_r, epnt, e_proxy, e_gate, e_bias, rpnt, r_proxy, r_gate, r_bias)


def _augment(emb, wcol):
    t = emb.shape[0]
    return jnp.concatenate(
        [emb, wcol.reshape(t, 1), jnp.zeros((t, _D2 - D - 1), jnp.float32)], axis=1)


def kernel(edge_index, edge_rel, ent_row, ent_col, rel_row, rel_col,
           ent_emb, rel_emb, e_gate, e_proxy, e_bias, e_attn,
           r_gate, r_proxy, r_bias, r_attn):
    src, dst = edge_index[0], edge_index[1]
    rhat = _normalize(rel_emb, axis=1)
    zeros_acc = jnp.zeros((N, _D2), jnp.float32)

    widoff = (jnp.arange(E, dtype=jnp.int32) // _EPW) * R

    def rep(t):
        return jnp.tile(t, (_NW, 1))

    tok = [zeros_acc]

    def chained_zeros():
        z, _ = lax.optimization_barrier((zeros_acc, tok[0]))
        return z

    zrtab = rep(_augment(jnp.zeros((R, D), jnp.float32),
                         jnp.ones((R,), jnp.float32)))
    rel2 = edge_rel + widoff
    acc_e = _sc_msg(ent_emb, zrtab, ent_col, ent_row, rel2, True,
                    chained_zeros())
    tok[0] = acc_e
    acc_r = _sc_msg(rep(rel_emb), zrtab, rel_col + widoff, rel_row, rel2,
                    True, chained_zeros())
    tok[0] = acc_r
    f0_e = _post(acc_e)
    f0_r = _post(acc_r)

    def dual(f0, attn):
        outs = [f0]
        feats = f0
        for l in range(DEPTH):
            a = jnp.squeeze(rhat @ attn[l], axis=-1)
            wtab = jnp.exp(a - jnp.max(a))
            reltab = rep(_augment(rhat, wtab))
            acc = _sc_msg(feats, reltab, src, dst, rel2, False,
                          chained_zeros())
            tok[0] = acc
            feats = _post(acc)
            outs.append(feats)
        return jnp.concatenate(outs, axis=1)

    out_e = dual(f0_e, e_attn)
    out_r = dual(f0_r, r_attn)
    return _tail(out_e, out_r, e_proxy, e_gate, e_bias, r_proxy, r_gate, r_bias)


# speedup vs baseline: 1.0243x; 1.0243x over previous
import functools

import jax
import jax.numpy as jnp
from jax import lax
from jax.experimental import pallas as pl
from jax.experimental.pallas import tpu as pltpu
from jax.experimental.pallas import tpu_sc as plsc

N = 10000
E = 320000
R = 1000
D = 128
DEPTH = 2
F3 = D * (DEPTH + 1)

_NC = 2
_NS = 16
_NW = _NC * _NS
_CH = 40
_EPW = E // _NW
_CPW = _EPW // _CH
_D2 = 144
_ZCH = 80
_NZC = N // _ZCH
_ZITER = (_NZC + _NS - 1) // _NS


def _normalize(x, axis):
    n = jnp.sqrt(jnp.sum(x * x, axis=axis, keepdims=True))
    return x / jnp.maximum(n, 1e-12)


def _sc_ids():
    cid = lax.axis_index("c")
    sid = lax.axis_index("s")
    return cid, sid, sid * _NC + cid


def _zero_acc(zeros_hbm, acc, sid):
    def zbody(j, carry):
        ch = sid + _NS * j

        @pl.when(ch < _NZC)
        def _():
            sl = pl.ds(ch * _ZCH, _ZCH)
            pltpu.sync_copy(zeros_hbm.at[sl], acc.at[sl])

        return carry

    lax.fori_loop(0, _ZITER, zbody, 0)
    plsc.subcore_barrier()


def _copy_out(acc, out, cid, sid):
    plsc.subcore_barrier()

    def obody(j, carry):
        ch = sid + _NS * j

        @pl.when(ch < _NZC)
        def _():
            sl = pl.ds(ch * _ZCH, _ZCH)
            pltpu.sync_copy(acc.at[sl], out.at[cid, sl])

        return carry

    lax.fori_loop(0, _ZITER, obody, 0)


_NB = 3
_NG = 2
_NBODY = 6
_LOOPC = (_CPW - 2) // _NBODY
_TAIL = _CPW - _NBODY * _LOOPC


def _msg_compute(xrows, rrows, orows):
    lanes = lax.iota(jnp.int32, 16)

    def gbody(g, gcarry):
        for kk in range(8):
            e = g * 8 + kk
            xs = [xrows[e, pl.ds(16 * q, 16)] for q in range(8)]
            rs = [rrows[e, pl.ds(16 * q, 16)] for q in range(9)]
            v = xs[0] * rs[0]
            for q in range(1, 8):
                v = v + xs[q] * rs[q]
            s = jnp.sum(v)
            w = rs[8][0]
            c2 = -2.0 * w * s
            for q in range(8):
                orows[e, pl.ds(16 * q, 16)] = w * xs[q] + c2 * rs[q]
            orows[e, pl.ds(D, 16)] = jnp.where(lanes == 0, w, 0.0)
        return gcarry

    lax.fori_loop(0, _CH // 8, gbody, 0)


def _copy_compute(xrows, orows):
    lanes = lax.iota(jnp.int32, 16)

    def gbody(g, gcarry):
        for kk in range(8):
            e = g * 8 + kk
            for q in range(8):
                orows[e, pl.ds(16 * q, 16)] = xrows[e, pl.ds(16 * q, 16)]
            orows[e, pl.ds(D, 16)] = jnp.where(lanes == 0, 1.0, 0.0)
        return gcarry

    lax.fori_loop(0, _CH // 8, gbody, 0)


def _msg_body(feats, reltab, sdr, mode, zeros, out, ibs, xrs, rrs, ors,
              mbuf, sgx, sgr, sss, acc):
    cid, sid, wid = _sc_ids()
    pltpu.sync_copy(mode, mbuf)
    plain = mbuf[...][0] != 0
    _zero_acc(zeros, acc, sid)

    def start_gather(t, p2, p3):
        pltpu.sync_copy(sdr.at[wid * _CPW + t], ibs[p3])
        pltpu.async_copy(feats.at[ibs[p3].at[0]], xrs[p2], sgx[p2])

        @pl.when(jnp.logical_not(plain))
        def _():
            pltpu.async_copy(reltab.at[ibs[p3].at[2]], rrs[p2], sgr[p2])

    def process(t, p2, p3, prefetch):
        def _wait_prev():
            q = (p3 + 1) % _NB
            pltpu.make_async_copy(ors[q], acc.at[ibs[q].at[1]], sss[q]).wait()

        if isinstance(t, int):
            if t >= 2:
                _wait_prev()
        else:
            pl.when(t >= 2)(_wait_prev)
        if prefetch:
            start_gather(t + 1, (p2 + 1) % _NG, (p3 + 1) % _NB)
        pltpu.make_async_copy(feats.at[ibs[p3].at[0]], xrs[p2], sgx[p2]).wait()

        @pl.when(plain)
        def _():
            _copy_compute(xrs[p2], ors[p3])

        @pl.when(jnp.logical_not(plain))
        def _():
            pltpu.make_async_copy(reltab.at[ibs[p3].at[2]], rrs[p2],
                                  sgr[p2]).wait()
            _msg_compute(xrs[p2], rrs[p2], ors[p3])

        pltpu.async_copy(ors[p3], acc.at[ibs[p3].at[1]], sss[p3], add=True)

    start_gather(0, 0, 0)

    def cbody(t6, carry):
        for b in range(_NBODY):
            process(t6 * _NBODY + b, b % _NG, b % _NB, True)
        return carry

    lax.fori_loop(0, _LOOPC, cbody, 0)
    t0 = _LOOPC * _NBODY
    for b in range(_TAIL):
        t = t0 + b
        process(t, t % _NG, t % _NB, b < _TAIL - 1)
    for t in (_CPW - 2, _CPW - 1):
        p = t % _NB
        pltpu.make_async_copy(ors[p], acc.at[ibs[p].at[1]], sss[p]).wait()
    _copy_out(acc, out, cid, sid)


_TPAD = N


def _sc_msg(feats, reltab, src, dst, rel, plain, zeros):
    def body(feats_, reltab_, sdr, mode, zeros_, out, i0, i1, i2, x0, x1,
             r0, r1, o0, o1, o2, mb, gx0, gx1, gr0, gr1,
             s0, s1, s2, acc):
        _msg_body(feats_, reltab_, sdr, mode, zeros_, out, [i0, i1, i2],
                  [x0, x1], [r0, r1], [o0, o1, o2], mb,
                  [gx0, gx1], [gr0, gr1], [s0, s1, s2], acc)

    k = functools.partial(
        pl.kernel,
        mesh=plsc.VectorSubcoreMesh(core_axis_name="c", subcore_axis_name="s"),
        out_type=jax.ShapeDtypeStruct((_NC, N, _D2), jnp.float32),
        compiler_params=pltpu.CompilerParams(use_tc_tiling_on_sc=False, needs_layout_passes=False),
        scratch_types=(
            [pltpu.VMEM((3, _CH), jnp.int32)] * _NB
            + [pltpu.VMEM((_CH, D), jnp.float32)] * _NG
            + [pltpu.VMEM((_CH, _D2), jnp.float32)] * _NG
            + [pltpu.VMEM((_CH, _D2), jnp.float32)] * _NB
            + [pltpu.VMEM((16,), jnp.int32)]
            + [pltpu.SemaphoreType.DMA] * (2 * _NG + _NB)
            + [pltpu.VMEM_SHARED((N, _D2), jnp.float32)]
        ),
    )(body)
    sdr = jnp.stack([src, dst, rel]).reshape(3, E // _CH, _CH)
    sdr = sdr.transpose(1, 0, 2)
    fpad = jnp.zeros((_TPAD, D), jnp.float32).at[:feats.shape[0]].set(feats)
    mode = jnp.full((16,), 1 if plain else 0, jnp.int32)
    return k(fpad, reltab, sdr, mode, zeros)


_PB = 1000


def _post_body(acc_ref, o_ref):
    a = acc_ref[0] + acc_ref[1]
    den = jnp.maximum(a[:, D:D + 1], 1e-30)
    o_ref[...] = jnp.tanh(a[:, :D] / den)


def _post(acc):
    return pl.pallas_call(
        _post_body,
        grid=(N // _PB,),
        in_specs=[pl.BlockSpec((_NC, _PB, _D2), lambda i: (0, i, 0))],
        out_specs=pl.BlockSpec((_PB, D), lambda i: (i, 0)),
        out_shape=jax.ShapeDtypeStruct((N, D), jnp.float32),
    )(acc)


def _tail_body(xe_ref, xr_ref, epnt_ref, ep_ref, eg_ref, eb_ref,
               rpnt_ref, rp_ref, rg_ref, rb_ref, out_ref):
    def one(x, pnt, p, g, b):
        xn = x * lax.rsqrt(jnp.maximum(jnp.sum(x * x, axis=1, keepdims=True), 1e-24))
        logits = jnp.dot(xn, pnt, preferred_element_type=jnp.float32)
        m = jnp.max(logits, axis=1, keepdims=True)
        ex = jnp.exp(logits - m)
        att = ex / jnp.sum(ex, axis=1, keepdims=True)
        pf = x - jnp.dot(att, p, preferred_element_type=jnp.float32)
        gate = jax.nn.sigmoid(jnp.dot(pf, g, preferred_element_type=jnp.float32) + b)
        return gate * x + (1.0 - gate) * pf

    out_ref[:, :F3] = one(xe_ref[...], epnt_ref[...], ep_ref[...], eg_ref[...], eb_ref[...])
    out_ref[:, F3:] = one(xr_ref[...], rpnt_ref[...], rp_ref[...], rg_ref[...], rb_ref[...])


def _tail(out_e, out_r, e_proxy, e_gate, e_bias, r_proxy, r_gate, r_bias):
    epnt = _normalize(e_proxy, axis=-1).T
    rpnt = _normalize(r_proxy, axis=-1).T
    row_spec = pl.BlockSpec((_PB, F3), lambda i: (i, 0))
    full = lambda shape: pl.BlockSpec(shape, lambda i: (0,) * len(shape))
    return pl.pallas_call(
        _tail_body,
        grid=(N // _PB,),
        in_specs=[
            row_spec, row_spec,
            full(epnt.shape), full(e_proxy.shape), full(e_gate.shape), full(e_bias.shape),
            full(rpnt.shape), full(r_proxy.shape), full(r_gate.shape), full(r_bias.shape),
        ],
        out_specs=pl.BlockSpec((_PB, 2 * F3), lambda i: (i, 0)),
        out_shape=jax.ShapeDtypeStruct((N, 2 * F3), jnp.float32),
    )(out_e, out_r, epnt, e_proxy, e_gate, e_bias, rpnt, r_proxy, r_gate, r_bias)


def _augment(emb, wcol):
    t = emb.shape[0]
    return jnp.concatenate(
        [emb, wcol.reshape(t, 1), jnp.zeros((t, _D2 - D - 1), jnp.float32)], axis=1)


def kernel(edge_index, edge_rel, ent_row, ent_col, rel_row, rel_col,
           ent_emb, rel_emb, e_gate, e_proxy, e_bias, e_attn,
           r_gate, r_proxy, r_bias, r_attn):
    src, dst = edge_index[0], edge_index[1]
    rhat = _normalize(rel_emb, axis=1)
    zeros_acc = jnp.zeros((N, _D2), jnp.float32)

    tok = [zeros_acc]

    def chained_zeros():
        z, _ = lax.optimization_barrier((zeros_acc, tok[0]))
        return z

    zrtab = _augment(jnp.zeros((R, D), jnp.float32),
                     jnp.ones((R,), jnp.float32))
    acc_e = _sc_msg(ent_emb, zrtab, ent_col, ent_row, edge_rel, True,
                    chained_zeros())
    tok[0] = acc_e
    acc_r = _sc_msg(rel_emb, zrtab, rel_col, rel_row, edge_rel,
                    True, chained_zeros())
    tok[0] = acc_r
    f0_e = _post(acc_e)
    f0_r = _post(acc_r)

    def dual(f0, attn):
        outs = [f0]
        feats = f0
        for l in range(DEPTH):
            a = jnp.squeeze(rhat @ attn[l], axis=-1)
            wtab = jnp.exp(a - jnp.max(a))
            reltab = _augment(rhat, wtab)
            acc = _sc_msg(feats, reltab, src, dst, edge_rel, False,
                          chained_zeros())
            tok[0] = acc
            feats = _post(acc)
            outs.append(feats)
        return jnp.concatenate(outs, axis=1)

    out_e = dual(f0_e, e_attn)
    out_r = dual(f0_r, r_attn)
    return _tail(out_e, out_r, e_proxy, e_gate, e_bias, r_proxy, r_gate, r_bias)
